# Initial kernel scaffold; baseline (speedup 1.0000x reference)
import jax
import jax.numpy as jnp
from jax.experimental import pallas as pl


def kernel(x, edge_index, edge_attr, batch, eps_0, W1_0, b1_0, W2_0, b2_0, eps_1, W1_1, b1_1, W2_1, b2_1, eps_2, W1_2, b1_2, W2_2, b2_2, lin1_W, lin1_b, lin2_W, lin2_b):
    raise NotImplementedError("write your pallas kernel here")


# speedup vs baseline: 5.6813x; 5.6813x over previous
import functools

import jax
import jax.numpy as jnp
from jax import lax
from jax.experimental import pallas as pl
from jax.experimental.pallas import tpu as pltpu
from jax.experimental.pallas import tpu_sc as plsc

_N_NODES = 10000
_N_EDGES = 320000
_D = 128
_N_GRAPHS = 128

_NC = 2
_NS = 16
_NW = _NC * _NS

_CHUNK = 128
_N_CHUNKS = _N_EDGES // _CHUNK
_FULL_ROUNDS = _N_CHUNKS // _NW
_REM = _N_CHUNKS - _FULL_ROUNDS * _NW
_ROWS_PER_SUB = 624
_ROWS_TAIL = _N_NODES - _NS * _ROWS_PER_SUB

_BLK = 1000

_sc_mesh = plsc.VectorSubcoreMesh(core_axis_name="c", subcore_axis_name="s")


@functools.partial(
    pl.kernel,
    out_type=jax.ShapeDtypeStruct((2 * _N_NODES, _D), jnp.float32),
    mesh=_sc_mesh,
    scratch_types=[
        pltpu.VMEM((_CHUNK,), jnp.int32),
        pltpu.VMEM((_CHUNK,), jnp.int32),
        pltpu.VMEM((_CHUNK, _D), jnp.float32),
        pltpu.VMEM_SHARED((_N_NODES, _D), jnp.float32),
        pltpu.SemaphoreType.DMA,
    ],
)
def _sc_segment_sum(x_hbm, src_hbm, dst_hbm, zeros_hbm, out_hbm,
                    src_v, dst_v, rows_v, acc_sh, sem):
    c = lax.axis_index("c")
    s = lax.axis_index("s")
    wid = c * _NS + s
    row0 = s * _ROWS_PER_SUB
    pltpu.sync_copy(zeros_hbm.at[pl.ds(row0, _ROWS_PER_SUB)],
                    acc_sh.at[pl.ds(row0, _ROWS_PER_SUB)])

    @pl.when(s == 0)
    def _():
        pltpu.sync_copy(zeros_hbm.at[pl.ds(_NS * _ROWS_PER_SUB, _ROWS_TAIL)],
                        acc_sh.at[pl.ds(_NS * _ROWS_PER_SUB, _ROWS_TAIL)])

    plsc.subcore_barrier()

    def _one_chunk(j):
        off = j * _CHUNK
        pltpu.sync_copy(src_hbm.at[pl.ds(off, _CHUNK)], src_v)
        pltpu.sync_copy(dst_hbm.at[pl.ds(off, _CHUNK)], dst_v)
        pltpu.async_copy(x_hbm.at[src_v], rows_v, sem).wait()
        pltpu.sync_copy(rows_v, acc_sh.at[dst_v], add=True)

    @pl.loop(0, _FULL_ROUNDS)
    def _(i):
        _one_chunk(i * _NW + wid)

    @pl.when(wid < _REM)
    def _():
        _one_chunk(_FULL_ROUNDS * _NW + wid)

    plsc.subcore_barrier()
    pltpu.sync_copy(acc_sh.at[pl.ds(row0, _ROWS_PER_SUB)],
                    out_hbm.at[pl.ds(c * _N_NODES + row0, _ROWS_PER_SUB)])

    @pl.when(s == 0)
    def _():
        pltpu.sync_copy(acc_sh.at[pl.ds(_NS * _ROWS_PER_SUB, _ROWS_TAIL)],
                        out_hbm.at[pl.ds(c * _N_NODES + _NS * _ROWS_PER_SUB, _ROWS_TAIL)])


def _gin_update(eps_ref, x_ref, a0_ref, a1_ref, W1_ref, b1_ref, W2_ref, b2_ref):
    h = (1.0 + eps_ref[0, 0]) * x_ref[...] + a0_ref[...] + a1_ref[...]
    h = jnp.maximum(
        jnp.dot(h, W1_ref[...], preferred_element_type=jnp.float32) + b1_ref[...], 0.0)
    h = jnp.maximum(
        jnp.dot(h, W2_ref[...], preferred_element_type=jnp.float32) + b2_ref[...], 0.0)
    return h


def _mlp_body(eps_ref, x_ref, a0_ref, a1_ref, W1_ref, b1_ref, W2_ref, b2_ref, o_ref):
    o_ref[...] = _gin_update(eps_ref, x_ref, a0_ref, a1_ref,
                             W1_ref, b1_ref, W2_ref, b2_ref)


def _mlp(eps, x, agg2, W1, b1, W2, b2):
    grid = _N_NODES // _BLK
    return pl.pallas_call(
        _mlp_body,
        grid=(grid,),
        in_specs=[
            pl.BlockSpec((1, 1), lambda i: (0, 0)),
            pl.BlockSpec((_BLK, _D), lambda i: (i, 0)),
            pl.BlockSpec((_BLK, _D), lambda i: (i, 0)),
            pl.BlockSpec((_BLK, _D), lambda i: (i + grid, 0)),
            pl.BlockSpec((_D, _D), lambda i: (0, 0)),
            pl.BlockSpec((1, _D), lambda i: (0, 0)),
            pl.BlockSpec((_D, _D), lambda i: (0, 0)),
            pl.BlockSpec((1, _D), lambda i: (0, 0)),
        ],
        out_specs=pl.BlockSpec((_BLK, _D), lambda i: (i, 0)),
        out_shape=jax.ShapeDtypeStruct((_N_NODES, _D), jnp.float32),
    )(eps.reshape(1, 1), x, agg2, agg2, W1, b1.reshape(1, _D), W2, b2.reshape(1, _D))


def _mlp_pool_body(eps_ref, batch_ref, x_ref, a0_ref, a1_ref,
                   W1_ref, b1_ref, W2_ref, b2_ref,
                   l1W_ref, l1b_ref, l2W_ref, l2b_ref, o_ref, acc_ref):
    i = pl.program_id(0)

    @pl.when(i == 0)
    def _():
        acc_ref[...] = jnp.zeros_like(acc_ref)

    h = _gin_update(eps_ref, x_ref, a0_ref, a1_ref, W1_ref, b1_ref, W2_ref, b2_ref)
    gids = batch_ref[0, 0, :]
    onehot = (gids[None, :] ==
              lax.broadcasted_iota(jnp.int32, (_N_GRAPHS, _BLK), 0)).astype(jnp.float32)
    acc_ref[...] += jnp.dot(onehot, h, preferred_element_type=jnp.float32)

    @pl.when(i == pl.num_programs(0) - 1)
    def _():
        pooled = acc_ref[...]
        y = jnp.maximum(
            jnp.dot(pooled, l1W_ref[...], preferred_element_type=jnp.float32)
            + l1b_ref[...], 0.0)
        o_ref[...] = jnp.sum(y * l2W_ref[...], axis=1, keepdims=True) + l2b_ref[0, 0]


def _mlp_pool(eps, batch3, x, agg2, W1, b1, W2, b2, l1W, l1b, l2W, l2b):
    grid = _N_NODES // _BLK
    return pl.pallas_call(
        _mlp_pool_body,
        grid=(grid,),
        in_specs=[
            pl.BlockSpec((1, 1), lambda i: (0, 0)),
            pl.BlockSpec((1, 1, _BLK), lambda i: (i, 0, 0)),
            pl.BlockSpec((_BLK, _D), lambda i: (i, 0)),
            pl.BlockSpec((_BLK, _D), lambda i: (i, 0)),
            pl.BlockSpec((_BLK, _D), lambda i: (i + grid, 0)),
            pl.BlockSpec((_D, _D), lambda i: (0, 0)),
            pl.BlockSpec((1, _D), lambda i: (0, 0)),
            pl.BlockSpec((_D, _D), lambda i: (0, 0)),
            pl.BlockSpec((1, _D), lambda i: (0, 0)),
            pl.BlockSpec((_D, _D), lambda i: (0, 0)),
            pl.BlockSpec((1, _D), lambda i: (0, 0)),
            pl.BlockSpec((1, _D), lambda i: (0, 0)),
            pl.BlockSpec((1, 1), lambda i: (0, 0)),
        ],
        out_specs=pl.BlockSpec((_N_GRAPHS, 1), lambda i: (0, 0)),
        out_shape=jax.ShapeDtypeStruct((_N_GRAPHS, 1), jnp.float32),
        scratch_shapes=[pltpu.VMEM((_N_GRAPHS, _D), jnp.float32)],
    )(eps.reshape(1, 1), batch3, x, agg2, agg2,
      W1, b1.reshape(1, _D), W2, b2.reshape(1, _D),
      l1W, l1b.reshape(1, _D), l2W.reshape(1, _D), l2b.reshape(1, 1))


def kernel(x, edge_index, edge_attr, batch,
           eps_0, W1_0, b1_0, W2_0, b2_0,
           eps_1, W1_1, b1_1, W2_1, b2_1,
           eps_2, W1_2, b1_2, W2_2, b2_2,
           lin1_W, lin1_b, lin2_W, lin2_b):
    ei = edge_index.astype(jnp.int32)
    src = ei[0]
    dst = ei[1]
    zeros = jnp.zeros((_N_NODES, _D), jnp.float32)
    batch3 = batch.astype(jnp.int32).reshape(_N_NODES // _BLK, 1, _BLK)

    agg2 = _sc_segment_sum(x, src, dst, zeros)
    h = _mlp(eps_0, x, agg2, W1_0, b1_0, W2_0, b2_0)
    agg2 = _sc_segment_sum(h, src, dst, zeros)
    h = _mlp(eps_1, h, agg2, W1_1, b1_1, W2_1, b2_1)
    agg2 = _sc_segment_sum(h, src, dst, zeros)
    return _mlp_pool(eps_2, batch3, h, agg2, W1_2, b1_2, W2_2, b2_2,
                     lin1_W, lin1_b, lin2_W, lin2_b)
